# Initial kernel scaffold; baseline (speedup 1.0000x reference)
import jax
import jax.numpy as jnp
from jax.experimental import pallas as pl


def kernel(input_ids, position_ids, word_table, pos_table, gamma, beta):
    raise NotImplementedError("write your pallas kernel here")


# speedup vs baseline: 1.4745x; 1.4745x over previous
import functools

import jax
import jax.numpy as jnp
from jax import lax
from jax.experimental import pallas as pl
from jax.experimental.pallas import tpu as pltpu
from jax.experimental.pallas import tpu_sc as plsc

EPS = 1e-12

NUM_SC_CORES = 2
NUM_SC_SUBCORES = 16
NUM_WORKERS = NUM_SC_CORES * NUM_SC_SUBCORES

CHUNK = 64


def _sc_gather_two(word_table, pos_table, ids, pids):
    bs = ids.shape[0]
    d = word_table.shape[1]
    per_w = bs // NUM_WORKERS
    assert per_w % CHUNK == 0 and per_w % 8 == 0

    mesh = plsc.VectorSubcoreMesh(core_axis_name="c", subcore_axis_name="s")
    out_sds = jax.ShapeDtypeStruct((bs, d), jnp.float32)

    @functools.partial(
        pl.kernel,
        out_type=[out_sds, out_sds],
        mesh=mesh,
        scratch_types=[
            pltpu.VMEM((per_w,), jnp.int32),
            pltpu.VMEM((per_w,), jnp.int32),
            pltpu.VMEM((CHUNK, d), jnp.float32),
            pltpu.VMEM((CHUNK, d), jnp.float32),
            pltpu.SemaphoreType.DMA,
            pltpu.SemaphoreType.DMA,
            pltpu.SemaphoreType.DMA,
            pltpu.SemaphoreType.DMA,
        ],
    )
    def sc_kernel(wt_hbm, pt_hbm, wid_hbm, pid_hbm, ow_hbm, op_hbm,
                  widx_v, pidx_v, wrows_v, prows_v, gsem_w, gsem_p,
                  ssem_w, ssem_p):
        wid = lax.axis_index("s") * NUM_SC_CORES + lax.axis_index("c")
        base = wid * per_w
        pltpu.sync_copy(wid_hbm.at[pl.ds(base, per_w)], widx_v)
        pltpu.sync_copy(pid_hbm.at[pl.ds(base, per_w)], pidx_v)

        @pl.loop(0, per_w, step=CHUNK)
        def _(off):
            gw = pltpu.async_copy(
                wt_hbm.at[widx_v.at[pl.ds(off, CHUNK)]], wrows_v, gsem_w)
            gp = pltpu.async_copy(
                pt_hbm.at[pidx_v.at[pl.ds(off, CHUNK)]], prows_v, gsem_p)
            gw.wait()
            gp.wait()
            sw = pltpu.async_copy(
                wrows_v, ow_hbm.at[pl.ds(base + off, CHUNK)], ssem_w)
            sp = pltpu.async_copy(
                prows_v, op_hbm.at[pl.ds(base + off, CHUNK)], ssem_p)
            sw.wait()
            sp.wait()

    return sc_kernel(word_table, pos_table, ids, pids)


def _ln_body(w_ref, p_ref, g_ref, b_ref, o_ref):
    x = w_ref[...] + p_ref[...]
    mean = jnp.mean(x, axis=-1, keepdims=True)
    xc = x - mean
    var = jnp.mean(xc * xc, axis=-1, keepdims=True)
    o_ref[...] = xc * lax.rsqrt(var + EPS) * g_ref[...] + b_ref[...]


def _tc_layernorm(w_rows, p_rows, gamma, beta):
    bs, d = w_rows.shape
    tw = 512
    grid = (bs // tw,)
    row_spec = pl.BlockSpec((tw, d), lambda i: (i, 0))
    vec_spec = pl.BlockSpec((1, d), lambda i: (0, 0))
    return pl.pallas_call(
        _ln_body,
        grid=grid,
        in_specs=[row_spec, row_spec, vec_spec, vec_spec],
        out_specs=row_spec,
        out_shape=jax.ShapeDtypeStruct((bs, d), jnp.float32),
    )(w_rows, p_rows, gamma.reshape(1, d), beta.reshape(1, d))


def kernel(input_ids, position_ids, word_table, pos_table, gamma, beta):
    b, s = input_ids.shape
    d = word_table.shape[1]
    ids = input_ids.reshape(-1)
    pids = position_ids.reshape(-1)
    w_rows, p_rows = _sc_gather_two(word_table, pos_table, ids, pids)
    out = _tc_layernorm(w_rows, p_rows, gamma, beta)
    return out.reshape(b, s, d)
